# baseline (speedup 1.0000x reference)
import functools

import jax
import jax.numpy as jnp
from jax.experimental import pallas as pl

B, K, N, H, D, KC = 16, 20, 1024, 4, 128, 16
IN_FEATS = 128
Nt = B * N
Kt = B * K
KP = 32
INV_SQRT_D = float(1.0 / (D ** 0.5))
BIG = 3.0e38


def _body(h_rec_ref, h0_ref, xr_ref, x0_ref,
          w_src_ref, wmlp_ref, b_ref,
          pos_ref, feat_ref):
    hb = h_rec_ref[...]
    xr3 = xr_ref[...]
    x03 = x0_ref[...]
    zpad = jnp.zeros((N, 5), jnp.float32)
    x_aug = jnp.concatenate([xr3, zpad], axis=1)
    srow_n = jax.lax.broadcasted_iota(jnp.int32, (8, N), 0)
    x_rec_t = jnp.where(srow_n == 3, 1.0, jnp.transpose(x_aug))
    x0_t = jnp.transpose(jnp.concatenate([x03, zpad], axis=1))

    hi = jax.lax.Precision.HIGHEST
    ft_src = jnp.dot(hb, w_src_ref[...], preferred_element_type=jnp.float32)
    ft_dst = jnp.dot(h0_ref[0], w_src_ref[...],
                     preferred_element_type=jnp.float32)
    ft_dstT = jnp.transpose(ft_dst)
    row = jax.lax.broadcasted_iota(jnp.int32, (H * D, KP), 0)
    cols = [jnp.where((row >= h * D) & (row < (h + 1) * D), ft_dstT, 0.0)
            for h in range(H)]
    bd = jnp.concatenate(cols, axis=1)
    e = jnp.exp(jnp.dot(ft_src, bd, preferred_element_type=jnp.float32)
                / jnp.sqrt(jnp.float32(D)))

    num = jnp.dot(x_rec_t, e, preferred_element_type=jnp.float32,
                  precision=hi)
    acc = jnp.zeros((8, KP), jnp.float32)
    for h in range(H):
        blk = num[:, h * KP:(h + 1) * KP]
        acc = acc + blk * (1.0 / blk[3:4, :])
    srow = jax.lax.broadcasted_iota(jnp.int32, (8, KP), 0)
    kp_pos_t = jnp.where(srow < 3, acc * (1.0 / H), 0.0)
    kp_pos = jnp.transpose(kp_pos_t)
    pos_ref[0] = kp_pos

    kpsq = jnp.sum(kp_pos * kp_pos, axis=1, keepdims=True)
    x0sq = jnp.sum(x0_t * x0_t, axis=0, keepdims=True)
    cross = jax.lax.dot_general(kp_pos_t, x0_t, (((0,), (0,)), ((), ())),
                                preferred_element_type=jnp.float32)
    d2 = kpsq + x0sq - 2.0 * cross

    lane = jax.lax.broadcasted_iota(jnp.int32, (KP, N), 1)
    lane8 = jax.lax.broadcasted_iota(jnp.int32, (KP, 8), 1)
    sel = jnp.zeros((KP, N), jnp.float32)
    dist_cols = []
    for _ in range(KC):
        mval = jnp.min(d2, axis=1, keepdims=True)
        idx = jnp.min(jnp.where(d2 == mval, lane, N), axis=1, keepdims=True)
        onehot = idx == lane
        ohf = jnp.where(onehot, 1.0, 0.0)
        sel = sel + ohf
        d2 = jnp.where(onehot, BIG, d2)
        xs = jnp.dot(ohf, x_aug, preferred_element_type=jnp.float32,
                     precision=hi)
        diff = jnp.where(lane8 < 3, xs - kp_pos, 0.0)
        dist_cols.append(jnp.sqrt(jnp.sum(diff * diff, axis=1, keepdims=True)))
    dists = jnp.concatenate(dist_cols, axis=1)

    h_m = jnp.dot(sel, hb, preferred_element_type=jnp.float32,
                  precision=hi) * (1.0 / KC)
    cat = jnp.concatenate([h_m, dists], axis=1)
    pre = (jnp.dot(cat, wmlp_ref[...], preferred_element_type=jnp.float32)
           + b_ref[...])
    feat_ref[0] = pre * jax.lax.logistic(pre)


@functools.partial(jax.jit, static_argnames=("interpret",))
def _run(h_rec, h0_kp, x_rec, x0_rec, W_src, W_mlp, b_mlp, interpret=False):
    f32 = jnp.float32
    h0_pad = jnp.pad(h0_kp.reshape(B, K, IN_FEATS),
                     ((0, 0), (0, KP - K), (0, 0)))
    b2 = b_mlp.reshape(1, D)

    pos, feat = pl.pallas_call(
        _body,
        grid=(B,),
        in_specs=[
            pl.BlockSpec((N, IN_FEATS), lambda b: (b, 0)),
            pl.BlockSpec((1, KP, IN_FEATS), lambda b: (b, 0, 0)),
            pl.BlockSpec((N, 3), lambda b: (b, 0)),
            pl.BlockSpec((N, 3), lambda b: (b, 0)),
            pl.BlockSpec((IN_FEATS, H * D), lambda b: (0, 0)),
            pl.BlockSpec((D + KC, D), lambda b: (0, 0)),
            pl.BlockSpec((1, D), lambda b: (0, 0)),
        ],
        out_specs=[
            pl.BlockSpec((1, KP, 8), lambda b: (b, 0, 0)),
            pl.BlockSpec((1, KP, D), lambda b: (b, 0, 0)),
        ],
        out_shape=[
            jax.ShapeDtypeStruct((B, KP, 8), f32),
            jax.ShapeDtypeStruct((B, KP, D), f32),
        ],
        interpret=interpret,
    )(h_rec, h0_pad, x_rec, x0_rec, W_src, W_mlp, b2)

    kp_pos = pos[:, :K, :3].reshape(Kt, 3)
    kp_feat = feat[:, :K, :].reshape(Kt, D)
    return kp_pos, kp_feat


def kernel(h_rec, h0_kp, x_rec, x0_rec, W_src, W_mlp, b_mlp,
           kp_batch_idx, edge_src, edge_dst):
    return _run(h_rec, h0_kp, x_rec, x0_rec, W_src, W_mlp, b_mlp)


# speedup vs baseline: 1.9773x; 1.9773x over previous
import functools

import jax
import jax.numpy as jnp
from jax.experimental import pallas as pl

B, K, N, H, D, KC = 16, 20, 1024, 4, 128, 16
IN_FEATS = 128
Nt = B * N
Kt = B * K
KP = 32
PB = 8
BIG = 3.0e38


def _body(h_rec_ref, h0_ref, xr_ref, x0_ref,
          w_src_ref, wmlp_ref, b_ref, pos_ref, feat_ref):
    hi = jax.lax.Precision.HIGHEST
    w_src = w_src_ref[...]
    w_mlp = wmlp_ref[...]
    bias = b_ref[...]
    G = range(PB)

    zpad = jnp.zeros((N, 5), jnp.float32)
    srow_n = jax.lax.broadcasted_iota(jnp.int32, (8, N), 0)
    hb = [h_rec_ref[pl.ds(i * N, N), :] for i in G]
    x_rec_t = [jnp.where(srow_n == 3, 1.0, jnp.transpose(jnp.concatenate(
        [xr_ref[pl.ds(i * N, N), :], zpad], axis=1))) for i in G]
    x0_t = [jnp.transpose(jnp.concatenate(
        [x0_ref[pl.ds(i * N, N), :], zpad], axis=1)) for i in G]

    ft_src = [jnp.dot(hb[i], w_src, preferred_element_type=jnp.float32)
              for i in G]
    ft_dst = [jnp.dot(h0_ref[i], w_src, preferred_element_type=jnp.float32)
              for i in G]
    row = jax.lax.broadcasted_iota(jnp.int32, (H * D, KP), 0)
    bd = []
    for i in G:
        ft_dstT = jnp.transpose(ft_dst[i])
        bd.append(jnp.concatenate(
            [jnp.where((row >= h * D) & (row < (h + 1) * D), ft_dstT, 0.0)
             for h in range(H)], axis=1))
    e = [jnp.exp(jnp.dot(ft_src[i], bd[i],
                         preferred_element_type=jnp.float32)
                 / jnp.sqrt(jnp.float32(D))) for i in G]

    num = [jnp.dot(x_rec_t[i], e[i], preferred_element_type=jnp.float32,
                   precision=hi) for i in G]
    srow = jax.lax.broadcasted_iota(jnp.int32, (8, KP), 0)
    kp_pos_t, kp_pos = [], []
    for i in G:
        acc = jnp.zeros((8, KP), jnp.float32)
        for h in range(H):
            blk = num[i][:, h * KP:(h + 1) * KP]
            acc = acc + blk * (1.0 / blk[3:4, :])
        pt = jnp.where(srow < 3, acc * (1.0 / H), 0.0)
        kp_pos_t.append(pt)
        kp_pos.append(jnp.transpose(pt))
        pos_ref[i] = kp_pos[i]

    d2, dx2 = [], []
    for i in G:
        kpsq = jnp.sum(kp_pos[i] * kp_pos[i], axis=1, keepdims=True)
        x0sq = jnp.sum(x0_t[i] * x0_t[i], axis=0, keepdims=True)
        cross = jax.lax.dot_general(kp_pos_t[i], x0_t[i],
                                    (((0,), (0,)), ((), ())),
                                    preferred_element_type=jnp.float32)
        d2.append(kpsq + x0sq - 2.0 * cross)
        a = jnp.zeros((KP, N), jnp.float32)
        for c in range(3):
            dc = x_rec_t[i][c:c + 1, :] - kp_pos[i][:, c:c + 1]
            a = a + dc * dc
        dx2.append(a)

    lane = jax.lax.broadcasted_iota(jnp.int32, (KP, N), 1)
    sel = [jnp.zeros((KP, N), jnp.float32) for _ in G]
    dist_cols = [[] for _ in G]
    for _ in range(KC):
        for i in G:
            mval = jnp.min(d2[i], axis=1, keepdims=True)
            idx = jnp.min(jnp.where(d2[i] == mval, lane, N),
                          axis=1, keepdims=True)
            onehot = idx == lane
            sel[i] = sel[i] + jnp.where(onehot, 1.0, 0.0)
            d2[i] = jnp.where(onehot, BIG, d2[i])
            dist_cols[i].append(jnp.sqrt(jnp.min(
                jnp.where(onehot, dx2[i], BIG), axis=1, keepdims=True)))

    for i in G:
        dists = jnp.concatenate(dist_cols[i], axis=1)
        h_m = jnp.dot(sel[i], hb[i], preferred_element_type=jnp.float32,
                      precision=hi) * (1.0 / KC)
        cat = jnp.concatenate([h_m, dists], axis=1)
        pre = (jnp.dot(cat, w_mlp, preferred_element_type=jnp.float32)
               + bias)
        feat_ref[i] = pre * jax.lax.logistic(pre)


@functools.partial(jax.jit, static_argnames=("interpret",))
def _run(h_rec, h0_kp, x_rec, x0_rec, W_src, W_mlp, b_mlp, interpret=False):
    f32 = jnp.float32
    h0_pad = jnp.pad(h0_kp.reshape(B, K, IN_FEATS),
                     ((0, 0), (0, KP - K), (0, 0)))
    b2 = b_mlp.reshape(1, D)

    pos, feat = pl.pallas_call(
        _body,
        grid=(B // PB,),
        in_specs=[
            pl.BlockSpec((PB * N, IN_FEATS), lambda b: (b, 0)),
            pl.BlockSpec((PB, KP, IN_FEATS), lambda b: (b, 0, 0)),
            pl.BlockSpec((PB * N, 3), lambda b: (b, 0)),
            pl.BlockSpec((PB * N, 3), lambda b: (b, 0)),
            pl.BlockSpec((IN_FEATS, H * D), lambda b: (0, 0)),
            pl.BlockSpec((D + KC, D), lambda b: (0, 0)),
            pl.BlockSpec((1, D), lambda b: (0, 0)),
        ],
        out_specs=[
            pl.BlockSpec((PB, KP, 8), lambda b: (b, 0, 0)),
            pl.BlockSpec((PB, KP, D), lambda b: (b, 0, 0)),
        ],
        out_shape=[
            jax.ShapeDtypeStruct((B, KP, 8), f32),
            jax.ShapeDtypeStruct((B, KP, D), f32),
        ],
        interpret=interpret,
    )(h_rec, h0_pad, x_rec, x0_rec, W_src, W_mlp, b2)

    kp_pos = pos[:, :K, :3].reshape(Kt, 3)
    kp_feat = feat[:, :K, :].reshape(Kt, D)
    return kp_pos, kp_feat


def kernel(h_rec, h0_kp, x_rec, x0_rec, W_src, W_mlp, b_mlp,
           kp_batch_idx, edge_src, edge_dst):
    return _run(h_rec, h0_kp, x_rec, x0_rec, W_src, W_mlp, b_mlp)
